# baseline (speedup 1.0000x reference)
import functools

import jax
import jax.numpy as jnp
from jax import lax
from jax.experimental import pallas as pl
from jax.experimental.pallas import tpu as pltpu

NSTREAMS = 8
BLOCK = 128


def _softmax_rows(logits):
    m = jnp.max(logits, axis=-1, keepdims=True)
    e = jnp.exp(logits - m)
    s = jnp.sum(e, axis=-1, keepdims=True)
    return e / s


def _gate_kernel(*refs):
    x_refs = refs[:NSTREAMS]
    w_ref, b_ref, probs_ref, ent_ref, w_scr, acc_ref = refs[NSTREAMS:]
    i = pl.program_id(0)
    nb = pl.num_programs(0)
    block = x_refs[0].shape[0]

    @pl.when(i == 0)
    def _init():
        w_scr[...] = w_ref[...].astype(jnp.bfloat16)
        acc_ref[0] = 0.0

    w = w_scr[...]
    bias = b_ref[...][None, :]
    total = jnp.zeros((), jnp.float32)
    for k, x_ref in enumerate(x_refs):
        p = _softmax_rows(lax.dot_general(
            x_ref[...].astype(jnp.bfloat16), w, (((1,), (1,)), ((), ())),
            preferred_element_type=jnp.float32) + bias)
        probs_ref[k * block:(k + 1) * block, :] = p
        total += jnp.sum(p * jnp.log(p + 1e-9))
    acc_ref[0] += total

    @pl.when(i == nb - 1)
    def _finalize():
        ent_ref[0] = -acc_ref[0] / (nb * NSTREAMS * block)


@jax.jit
def _switch_gate(x, W, b):
    tokens, in_dim = x.shape
    num_experts = W.shape[0]
    step_rows = NSTREAMS * BLOCK
    nb = tokens // step_rows

    def _xspec(k):
        return pl.BlockSpec((BLOCK, in_dim), lambda i, k=k: (NSTREAMS * i + k, 0))

    probs, ent = pl.pallas_call(
        _gate_kernel,
        grid=(nb,),
        in_specs=[_xspec(k) for k in range(NSTREAMS)] + [
            pl.BlockSpec((num_experts, in_dim), lambda i: (0, 0)),
            pl.BlockSpec((num_experts,), lambda i: (0,)),
        ],
        out_specs=[
            pl.BlockSpec((step_rows, num_experts), lambda i: (i, 0)),
            pl.BlockSpec(memory_space=pltpu.SMEM),
        ],
        out_shape=[
            jax.ShapeDtypeStruct((tokens, num_experts), jnp.float32),
            jax.ShapeDtypeStruct((1,), jnp.float32),
        ],
        scratch_shapes=[
            pltpu.VMEM((num_experts, in_dim), jnp.bfloat16),
            pltpu.SMEM((1,), jnp.float32),
        ],
        compiler_params=pltpu.CompilerParams(
            dimension_semantics=("arbitrary",),
        ),
    )(*([x] * NSTREAMS), W, b)
    return probs, ent[0]


def kernel(x, W, b):
    return _switch_gate(x, W, b)


# speedup vs baseline: 1.0014x; 1.0014x over previous
import functools

import jax
import jax.numpy as jnp
from jax import lax
from jax.experimental import pallas as pl
from jax.experimental.pallas import tpu as pltpu

NSTREAMS = 2
BLOCK = 512


def _softmax_rows(logits):
    m = jnp.max(logits, axis=-1, keepdims=True)
    e = jnp.exp(logits - m)
    s = jnp.sum(e, axis=-1, keepdims=True)
    return e / s


def _gate_kernel(*refs):
    x_refs = refs[:NSTREAMS]
    w_ref, b_ref, probs_ref, ent_ref, w_scr, acc_ref = refs[NSTREAMS:]
    i = pl.program_id(0)
    nb = pl.num_programs(0)
    block = x_refs[0].shape[0]

    @pl.when(i == 0)
    def _init():
        w_scr[...] = w_ref[...].astype(jnp.bfloat16)
        acc_ref[0] = 0.0

    w = w_scr[...]
    bias = b_ref[...][None, :]
    total = jnp.zeros((), jnp.float32)
    for k, x_ref in enumerate(x_refs):
        p = _softmax_rows(lax.dot_general(
            x_ref[...].astype(jnp.bfloat16), w, (((1,), (1,)), ((), ())),
            preferred_element_type=jnp.float32) + bias)
        probs_ref[k * block:(k + 1) * block, :] = p
        total += jnp.sum(p * jnp.log(p + 1e-9))
    acc_ref[0] += total

    @pl.when(i == nb - 1)
    def _finalize():
        ent_ref[0] = -acc_ref[0] / (nb * NSTREAMS * block)


@jax.jit
def _switch_gate(x, W, b):
    tokens, in_dim = x.shape
    num_experts = W.shape[0]
    step_rows = NSTREAMS * BLOCK
    nb = tokens // step_rows

    def _xspec(k):
        return pl.BlockSpec((BLOCK, in_dim), lambda i, k=k: (NSTREAMS * i + k, 0))

    probs, ent = pl.pallas_call(
        _gate_kernel,
        grid=(nb,),
        in_specs=[_xspec(k) for k in range(NSTREAMS)] + [
            pl.BlockSpec((num_experts, in_dim), lambda i: (0, 0)),
            pl.BlockSpec((num_experts,), lambda i: (0,)),
        ],
        out_specs=[
            pl.BlockSpec((step_rows, num_experts), lambda i: (i, 0)),
            pl.BlockSpec(memory_space=pltpu.SMEM),
        ],
        out_shape=[
            jax.ShapeDtypeStruct((tokens, num_experts), jnp.float32),
            jax.ShapeDtypeStruct((1,), jnp.float32),
        ],
        scratch_shapes=[
            pltpu.VMEM((num_experts, in_dim), jnp.bfloat16),
            pltpu.SMEM((1,), jnp.float32),
        ],
        compiler_params=pltpu.CompilerParams(
            dimension_semantics=("arbitrary",),

        ),
    )(*([x] * NSTREAMS), W, b)
    return probs, ent[0]


def kernel(x, W, b):
    return _switch_gate(x, W, b)
